# Initial kernel scaffold; baseline (speedup 1.0000x reference)
import jax
import jax.numpy as jnp
from jax.experimental import pallas as pl


def kernel(x, prototypes):
    raise NotImplementedError("write your pallas kernel here")


# speedup vs baseline: 1.5420x; 1.5420x over previous
import functools

import jax
import jax.numpy as jnp
from jax import lax
from jax.experimental import pallas as pl
from jax.experimental.pallas import tpu as pltpu

K = 1024
D = 256
N = 16384
NC = 512


def _body(pn_ref, pp_ref, xn_ref, xx_ref, out_ref, sums_ref, cnts_ref):
    i = pl.program_id(0)
    nblocks = pl.num_programs(0)

    @pl.when(i == 0)
    def _init():
        sums_ref[...] = jnp.zeros_like(sums_ref)
        cnts_ref[...] = jnp.zeros_like(cnts_ref)

    pn = pn_ref[...]
    xn = xn_ref[...]
    pp = pp_ref[...]
    xx = xx_ref[...]

    s = lax.dot_general(pn, xn, (((1,), (1,)), ((), ())),
                        preferred_element_type=jnp.float32)
    d2 = (pp + xx) - 2.0 * s
    dist = jnp.sqrt(jnp.maximum(d2, 0.0))

    m = jnp.min(dist, axis=0, keepdims=True)
    iota_k = lax.broadcasted_iota(jnp.int32, (K, NC), 0)
    sel = jnp.where(dist == m, iota_k, K)
    idx = jnp.min(sel, axis=0, keepdims=True)

    onehot = jnp.where(iota_k == idx, 1.0, 0.0)

    sums_ref[...] += lax.dot_general(
        onehot, xn, (((1,), (0,)), ((), ())),
        preferred_element_type=jnp.float32,
        precision=lax.Precision.HIGHEST)
    cnts_ref[...] += jnp.sum(onehot, axis=1, keepdims=True)

    @pl.when(i == nblocks - 1)
    def _fin():
        cnts = cnts_ref[...]
        sums = sums_ref[...]
        out_ref[...] = jnp.where(cnts > 0.0,
                                 sums / jnp.maximum(cnts, 1.0),
                                 jnp.zeros_like(sums))


@functools.partial(jax.jit, static_argnames=("interpret",))
def _h2t(x, prototypes, interpret=False):
    pn = prototypes / jnp.linalg.norm(prototypes, axis=1)[:, None]
    xn = (x / jnp.linalg.norm(x, axis=-1)[..., None])[0]
    pp = jnp.sum(pn * pn, axis=1)[:, None]
    xx = jnp.sum(xn * xn, axis=1)[None, :]

    grid = N // NC
    out = pl.pallas_call(
        _body,
        grid=(grid,),
        in_specs=[
            pl.BlockSpec((K, D), lambda i: (0, 0)),
            pl.BlockSpec((K, 1), lambda i: (0, 0)),
            pl.BlockSpec((NC, D), lambda i: (i, 0)),
            pl.BlockSpec((1, NC), lambda i: (0, i)),
        ],
        out_specs=pl.BlockSpec((K, D), lambda i: (0, 0)),
        out_shape=jax.ShapeDtypeStruct((K, D), jnp.float32),
        scratch_shapes=[
            pltpu.VMEM((K, D), jnp.float32),
            pltpu.VMEM((K, 1), jnp.float32),
        ],
        compiler_params=pltpu.CompilerParams(
            dimension_semantics=("arbitrary",)),
        interpret=interpret,
    )(pn, pp, xn, xx)
    return out.reshape(1, K * D)


def kernel(x, prototypes):
    return _h2t(x, prototypes)
